# baseline (device time: 99838 ns/iter reference)
import jax
import jax.numpy as jnp
from jax import lax
from jax.experimental import pallas as pl
from jax.experimental.pallas import tpu as pltpu

N_DEV = 4


def kernel(A, B):
    m, k = A.shape
    _, n = B.shape
    ch = m // N_DEV
    h = n // 2
    q = n // 4

    def body(
        a_hbm,
        b_hbm,
        out_ref,
        send_ref,
        recv_ref,
        astage_ref,
        a16_ref,
        bstage_ref,
        b16_ref,
        send_sems,
        recv_sems,
        adma_sem,
        bdma_sem,
    ):
        my = lax.axis_index("i")
        right = (my + 1) % N_DEV
        left = (my + 3) % N_DEV
        diag = (my + 2) % N_DEV

        barrier = pltpu.get_barrier_semaphore()
        for nbr in (left, right):
            pl.semaphore_signal(
                barrier, inc=1, device_id=(nbr,),
                device_id_type=pl.DeviceIdType.MESH,
            )
        pl.semaphore_wait(barrier, 2)

        def fetch_a(rows):
            return pltpu.make_async_copy(
                a_hbm.at[pl.ds(rows * ch, ch), :], astage_ref, adma_sem
            )

        def fetch_b(j):
            return pltpu.make_async_copy(
                b_hbm.at[:, pl.ds(j * q, q)], bstage_ref, bdma_sem
            )

        def cast_b(j):
            b16_ref[:, pl.ds(j * q, q)] = bstage_ref[:, :].astype(jnp.bfloat16)

        def mmq(slot, j):
            return jnp.dot(
                a16_ref[slot, :, :],
                b16_ref[:, pl.ds(j * q, q)],
                preferred_element_type=jnp.float32,
            ).astype(jnp.bfloat16)

        qa = pl.ds(0, q)
        qb = pl.ds(q, q)
        L = slice(0, h)
        R = slice(h, n)

        def make(slot, half_q, sem, tgt):
            cols = qa if half_q == 0 else qb
            return pltpu.make_async_remote_copy(
                src_ref=send_ref.at[slot, :, cols],
                dst_ref=recv_ref.at[slot, :, cols],
                send_sem=send_sems.at[sem],
                recv_sem=recv_sems.at[sem],
                device_id=(tgt,),
                device_id_type=pl.DeviceIdType.MESH,
            )

        rd0 = [make(0, i, 0 + i, right) for i in range(2)]
        rd1 = [make(1, i, 2 + i, right) for i in range(2)]
        rd2 = [make(2, i, 4 + i, right) for i in range(2)]
        rd3 = [make(3, i, 6 + i, left) for i in range(2)]
        rd4 = [make(4, i, 8 + i, left) for i in range(2)]
        rd5 = [make(5, i, 10 + i, left) for i in range(2)]

        fa = fetch_a(diag)
        fa.start()
        fb = fetch_b(0)
        fb.start()
        fa.wait()
        a16_ref[0] = astage_ref[:, :].astype(jnp.bfloat16)
        fa = fetch_a(right)
        fa.start()
        fb.wait()
        cast_b(0)
        fb = fetch_b(2)
        fb.start()

        send_ref[0, :, qa] = mmq(0, 0)
        rd0[0].start()
        fb.wait()
        cast_b(2)
        fb = fetch_b(1)
        fb.start()
        send_ref[3, :, qa] = mmq(0, 2)
        rd3[0].start()
        fb.wait()
        cast_b(1)
        fb = fetch_b(3)
        fb.start()
        send_ref[0, :, qb] = mmq(0, 1)
        rd0[1].start()
        fb.wait()
        cast_b(3)
        fa.wait()
        a16_ref[1] = astage_ref[:, :].astype(jnp.bfloat16)
        fa = fetch_a(left)
        fa.start()
        send_ref[3, :, qb] = mmq(0, 3)
        rd3[1].start()

        send_ref[1, :, qa] = mmq(1, 2)
        rd1[0].start()
        fa.wait()
        a16_ref[0] = astage_ref[:, :].astype(jnp.bfloat16)
        fa = fetch_a(my)
        fa.start()
        send_ref[4, :, qa] = mmq(0, 0)
        rd4[0].start()
        send_ref[1, :, qb] = mmq(1, 3)
        rd1[1].start()
        send_ref[4, :, qb] = mmq(0, 1)
        rd4[1].start()

        send_ref[2, :, qa] = mmq(1, 0)
        send_ref[5, :, qa] = mmq(0, 2)
        send_ref[2, :, qb] = mmq(1, 1)
        send_ref[5, :, qb] = mmq(0, 3)

        rd0[0].wait_recv()
        send_ref[2, :, qa] = send_ref[2, :, qa] + recv_ref[0, :, qa]
        rd2[0].start()
        rd3[0].wait_recv()
        send_ref[5, :, qa] = send_ref[5, :, qa] + recv_ref[3, :, qa]
        rd5[0].start()
        rd0[1].wait_recv()
        send_ref[2, :, qb] = send_ref[2, :, qb] + recv_ref[0, :, qb]
        rd2[1].start()
        rd3[1].wait_recv()
        send_ref[5, :, qb] = send_ref[5, :, qb] + recv_ref[3, :, qb]
        rd5[1].start()

        fa.wait()
        a16_ref[1] = astage_ref[:, :].astype(jnp.bfloat16)
        out_ref[:, L] = jnp.dot(
            a16_ref[1, :, :], b16_ref[:, L],
            preferred_element_type=jnp.float32,
        ).astype(jnp.bfloat16)
        out_ref[:, R] = jnp.dot(
            a16_ref[1, :, :], b16_ref[:, R],
            preferred_element_type=jnp.float32,
        ).astype(jnp.bfloat16)

        rd4[0].wait_recv()
        rd4[1].wait_recv()
        rd2[0].wait_recv()
        rd2[1].wait_recv()
        out_ref[:, L] = out_ref[:, L] + (recv_ref[2] + recv_ref[4])
        rd1[0].wait_recv()
        rd1[1].wait_recv()
        rd5[0].wait_recv()
        rd5[1].wait_recv()
        out_ref[:, R] = out_ref[:, R] + (recv_ref[5] + recv_ref[1])

        for pair in (rd0, rd1, rd2, rd3, rd4, rd5):
            pair[0].wait_send()
            pair[1].wait_send()

    return pl.pallas_call(
        body,
        out_shape=jax.ShapeDtypeStruct((ch, n), jnp.bfloat16),
        in_specs=[
            pl.BlockSpec(memory_space=pltpu.MemorySpace.HBM),
            pl.BlockSpec(memory_space=pltpu.MemorySpace.HBM),
        ],
        out_specs=pl.BlockSpec(memory_space=pltpu.VMEM),
        scratch_shapes=[
            pltpu.VMEM((6, ch, h), jnp.bfloat16),
            pltpu.VMEM((6, ch, h), jnp.bfloat16),
            pltpu.VMEM((ch, k), jnp.float32),
            pltpu.VMEM((2, ch, k), jnp.bfloat16),
            pltpu.VMEM((k, q), jnp.float32),
            pltpu.VMEM((k, n), jnp.bfloat16),
            pltpu.SemaphoreType.DMA((12,)),
            pltpu.SemaphoreType.DMA((12,)),
            pltpu.SemaphoreType.DMA,
            pltpu.SemaphoreType.DMA,
        ],
        compiler_params=pltpu.CompilerParams(
            vmem_limit_bytes=110 * 1024 * 1024,
            collective_id=0,
        ),
    )(A, B)


# device time: 99711 ns/iter; 1.0013x vs baseline; 1.0013x over previous
import jax
import jax.numpy as jnp
from jax import lax
from jax.experimental import pallas as pl
from jax.experimental.pallas import tpu as pltpu

N_DEV = 4


def kernel(A, B):
    m, k = A.shape
    _, n = B.shape
    ch = m // N_DEV
    h = n // 2
    q = n // 4

    def body(
        a_hbm,
        b_hbm,
        out_ref,
        send_ref,
        recv_ref,
        astage_ref,
        a16_ref,
        bstage_ref,
        b16_ref,
        send_sems,
        recv_sems,
        adma_sem,
        bdma_sem,
    ):
        my = lax.axis_index("i")
        right = (my + 1) % N_DEV
        left = (my + 3) % N_DEV
        diag = (my + 2) % N_DEV

        barrier = pltpu.get_barrier_semaphore()
        for nbr in (left, right):
            pl.semaphore_signal(
                barrier, inc=1, device_id=(nbr,),
                device_id_type=pl.DeviceIdType.MESH,
            )
        pl.semaphore_wait(barrier, 2)

        def fetch_a(rows):
            return pltpu.make_async_copy(
                a_hbm.at[pl.ds(rows * ch, ch), :], astage_ref, adma_sem
            )

        def fetch_b(j):
            return pltpu.make_async_copy(
                b_hbm.at[:, pl.ds(j * q, q)], bstage_ref, bdma_sem
            )

        def cast_b(j):
            b16_ref[:, pl.ds(j * q, q)] = bstage_ref[:, :].astype(jnp.bfloat16)

        def mm(slot, cols):
            return jnp.dot(
                a16_ref[slot, :, :],
                b16_ref[:, cols],
                preferred_element_type=jnp.float32,
            ).astype(jnp.bfloat16)

        def make(slot, tgt, sem, cols=None):
            src = send_ref.at[slot] if cols is None else send_ref.at[slot, :, cols]
            dst = recv_ref.at[slot] if cols is None else recv_ref.at[slot, :, cols]
            return pltpu.make_async_remote_copy(
                src_ref=src,
                dst_ref=dst,
                send_sem=send_sems.at[sem],
                recv_sem=recv_sems.at[sem],
                device_id=(tgt,),
                device_id_type=pl.DeviceIdType.MESH,
            )

        L = slice(0, h)
        R = slice(h, n)
        qa = pl.ds(0, q)
        qb = pl.ds(q, q)

        rd0a = make(0, right, 0, qa)
        rd0b = make(0, right, 6, qb)
        rd3a = make(3, left, 3, qa)
        rd3b = make(3, left, 7, qb)
        rd1 = make(1, right, 1)
        rd2 = make(2, right, 2)
        rd4 = make(4, left, 4)
        rd5 = make(5, left, 5)

        fa = fetch_a(diag)
        fa.start()
        fb = fetch_b(0)
        fb.start()
        fa.wait()
        a16_ref[0] = astage_ref[:, :].astype(jnp.bfloat16)
        fb.wait()
        cast_b(0)
        fb = fetch_b(2)
        fb.start()

        send_ref[0, :, qa] = mm(0, pl.ds(0, q))
        rd0a.start()
        fb.wait()
        cast_b(2)
        fb = fetch_b(1)
        fb.start()
        send_ref[3, :, qa] = mm(0, pl.ds(h, q))
        rd3a.start()
        fb.wait()
        cast_b(1)
        fb = fetch_b(3)
        fb.start()
        fa = fetch_a(right)
        fa.start()
        send_ref[0, :, qb] = mm(0, pl.ds(q, q))
        rd0b.start()
        fb.wait()
        cast_b(3)
        send_ref[3, :, qb] = mm(0, pl.ds(h + q, q))
        rd3b.start()

        fa.wait()
        a16_ref[1] = astage_ref[:, :].astype(jnp.bfloat16)
        fa = fetch_a(left)
        fa.start()
        send_ref[1] = mm(1, R)
        rd1.start()
        send_ref[2] = mm(1, L)

        fa.wait()
        a16_ref[0] = astage_ref[:, :].astype(jnp.bfloat16)
        fa = fetch_a(my)
        fa.start()
        send_ref[4] = mm(0, L)
        rd4.start()
        send_ref[5] = mm(0, R)

        rd0a.wait_recv()
        rd0b.wait_recv()
        send_ref[2] = send_ref[2] + recv_ref[0]
        rd2.start()
        rd3a.wait_recv()
        rd3b.wait_recv()
        send_ref[5] = send_ref[5] + recv_ref[3]
        rd5.start()

        fa.wait()
        a16_ref[1] = astage_ref[:, :].astype(jnp.bfloat16)
        out_ref[:, L] = mm(1, L)
        out_ref[:, R] = mm(1, R)

        rd4.wait_recv()
        rd2.wait_recv()
        out_ref[:, L] = out_ref[:, L] + (recv_ref[2] + recv_ref[4])
        rd1.wait_recv()
        rd5.wait_recv()
        out_ref[:, R] = out_ref[:, R] + (recv_ref[5] + recv_ref[1])

        for r in (rd0a, rd0b, rd3a, rd3b, rd1, rd2, rd4, rd5):
            r.wait_send()

    return pl.pallas_call(
        body,
        out_shape=jax.ShapeDtypeStruct((ch, n), jnp.bfloat16),
        in_specs=[
            pl.BlockSpec(memory_space=pltpu.MemorySpace.HBM),
            pl.BlockSpec(memory_space=pltpu.MemorySpace.HBM),
        ],
        out_specs=pl.BlockSpec(memory_space=pltpu.VMEM),
        scratch_shapes=[
            pltpu.VMEM((6, ch, h), jnp.bfloat16),
            pltpu.VMEM((6, ch, h), jnp.bfloat16),
            pltpu.VMEM((ch, k), jnp.float32),
            pltpu.VMEM((2, ch, k), jnp.bfloat16),
            pltpu.VMEM((k, q), jnp.float32),
            pltpu.VMEM((k, n), jnp.bfloat16),
            pltpu.SemaphoreType.DMA((8,)),
            pltpu.SemaphoreType.DMA((8,)),
            pltpu.SemaphoreType.DMA,
            pltpu.SemaphoreType.DMA,
        ],
        compiler_params=pltpu.CompilerParams(
            vmem_limit_bytes=110 * 1024 * 1024,
            collective_id=0,
        ),
    )(A, B)
